# initial kernel scaffold (unmeasured)
import jax
import jax.numpy as jnp
from jax import lax
from jax.experimental import pallas as pl
from jax.experimental.pallas import tpu as pltpu


def kernel(
    x,
):
    def body(*refs):
        pass

    out_shape = jax.ShapeDtypeStruct(..., jnp.float32)
    return pl.pallas_call(body, out_shape=out_shape)(...)


# baseline (device time: 169216 ns/iter reference)
import jax
import jax.numpy as jnp
from jax import lax
from jax.experimental import pallas as pl
from jax.experimental.pallas import tpu as pltpu

N_DEV = 4


def kernel(x):
    m_per, n = x.shape
    hm = m_per // 2
    xb = x.astype(jnp.bfloat16)

    def body(x_ref, out_ref, send_sems, recv_sems, copy_sem):
        my_pos = lax.axis_index("i")
        left = (my_pos - 1) % N_DEV
        right = (my_pos + 1) % N_DEV
        opp = (my_pos + 2) % N_DEV

        barrier_sem = pltpu.get_barrier_semaphore()
        for nbr in [left, right]:
            pl.semaphore_signal(
                barrier_sem, inc=1,
                device_id=(nbr,), device_id_type=pl.DeviceIdType.MESH,
            )
        pl.semaphore_wait(barrier_sem, 2)

        own_copy = pltpu.make_async_copy(
            x_ref, out_ref.at[pl.ds(my_pos * m_per, m_per)], copy_sem
        )
        own_copy.start()

        p1r = pltpu.make_async_remote_copy(
            src_ref=x_ref,
            dst_ref=out_ref.at[pl.ds(my_pos * m_per, m_per)],
            send_sem=send_sems.at[0],
            recv_sem=recv_sems.at[0],
            device_id=(right,),
            device_id_type=pl.DeviceIdType.MESH,
        )
        p1l = pltpu.make_async_remote_copy(
            src_ref=x_ref,
            dst_ref=out_ref.at[pl.ds(my_pos * m_per, m_per)],
            send_sem=send_sems.at[1],
            recv_sem=recv_sems.at[1],
            device_id=(left,),
            device_id_type=pl.DeviceIdType.MESH,
        )
        p1r.start()
        p1l.start()
        p1r.wait()
        p1l.wait()

        p2r = pltpu.make_async_remote_copy(
            src_ref=out_ref.at[pl.ds(left * m_per, hm)],
            dst_ref=out_ref.at[pl.ds(left * m_per, hm)],
            send_sem=send_sems.at[2],
            recv_sem=recv_sems.at[2],
            device_id=(right,),
            device_id_type=pl.DeviceIdType.MESH,
        )
        p2l = pltpu.make_async_remote_copy(
            src_ref=out_ref.at[pl.ds(right * m_per + hm, hm)],
            dst_ref=out_ref.at[pl.ds(right * m_per + hm, hm)],
            send_sem=send_sems.at[3],
            recv_sem=recv_sems.at[3],
            device_id=(left,),
            device_id_type=pl.DeviceIdType.MESH,
        )
        p2r.start()
        p2l.start()
        p2r.wait()
        p2l.wait()

        own_copy.wait()

    return pl.pallas_call(
        body,
        out_shape=jax.ShapeDtypeStruct((N_DEV * m_per, n), jnp.bfloat16),
        in_specs=[pl.BlockSpec(memory_space=pltpu.MemorySpace.HBM)],
        out_specs=pl.BlockSpec(memory_space=pltpu.MemorySpace.HBM),
        scratch_shapes=[
            pltpu.SemaphoreType.DMA((4,)),
            pltpu.SemaphoreType.DMA((4,)),
            pltpu.SemaphoreType.DMA,
        ],
        compiler_params=pltpu.CompilerParams(collective_id=0),
    )(xb)
